# baseline (speedup 1.0000x reference)
import jax
import jax.numpy as jnp
from jax import lax
from jax.experimental import pallas as pl
from jax.experimental.pallas import tpu as pltpu
from jax.experimental.pallas import tpu_sc as plsc

N = 100000
E = 3200000
T = 10
NTYPES = 3

LANES = 16
NTILES = 16
NCORES = 2
NP = 100096
SL = NP // NTILES
NVREG = SL // LANES
RPT = 784
CR = 28
NCHUNK = RPT // CR
ER = NCORES * NTILES * RPT
EP = ER * 128
CE = CR * 128
ET = RPT * 128
NPAD_SLOTS = NP - N


def _sc_kernel_body(susc_hbm, trans_hbm, betas_hbm, idx1_hbm, src_hbm,
                    dst_hbm,
                    out_hbm, partg_hbm, partp_hbm,
                    trans3, accg, accp,
                    suscb, cumb, invb, zbuf, na, nb, bbuf,
                    e0a, e0b, e1a, e1b, e2a, e2b,
                    sem0, sem1, gsem, ssem, esem0, esem1, xsem):
    cid = lax.axis_index("c")
    tid = lax.axis_index("s")
    s0 = tid * SL
    wid = cid * NTILES + tid
    eb0 = wid * ET
    my_part = cid * NP
    other_part = (1 - cid) * NP

    def cross_core_sync():
        pltpu.semaphore_signal(xsem, 1, core_index=1 - cid)
        pltpu.semaphore_wait(xsem, 1)

    def edge_pass(gidx_hbm, sidx_hbm, table, acc, ld0=None, ld1=None):
        ebufs0 = (e0a, e0b)
        ebufs1 = (e1a, e1b)
        ebufs2 = (e2a, e2b)
        if ld0 is None:
            ld0 = pltpu.async_copy(gidx_hbm.at[pl.ds(eb0, CE)], e0a, sem0)
            ld1 = pltpu.async_copy(sidx_hbm.at[pl.ds(eb0, CE)], e1a, sem1)
        ld0.wait()
        g = pltpu.async_copy(table.at[e0a], e2a, gsem)
        ld1.wait()
        ld0n = ld1n = None
        if NCHUNK > 1:
            rr = eb0 + CE
            ld0n = pltpu.async_copy(gidx_hbm.at[pl.ds(rr, CE)], e0b, sem0)
            ld1n = pltpu.async_copy(sidx_hbm.at[pl.ds(rr, CE)], e1b, sem1)
        for c in range(NCHUNK):
            b = c % 2
            g.wait()
            s = pltpu.async_copy(ebufs2[b], acc.at[ebufs1[b]], ssem,
                                 add=True)
            if c + 1 < NCHUNK:
                ld0n.wait()
                g = pltpu.async_copy(table.at[ebufs0[1 - b]],
                                     ebufs2[1 - b], gsem)
                if c + 2 < NCHUNK:
                    rr = eb0 + (c + 2) * CE
                    ld0n = pltpu.async_copy(gidx_hbm.at[pl.ds(rr, CE)],
                                            ebufs0[b], sem0)
            s.wait()
            if c + 1 < NCHUNK:
                ld1n.wait()
                if c + 2 < NCHUNK:
                    rr = eb0 + (c + 2) * CE
                    ld1n = pltpu.async_copy(sidx_hbm.at[pl.ds(rr, CE)],
                                            ebufs1[b], sem1)

    def zfill(j, _):
        z = jnp.zeros((LANES,), jnp.float32)
        zbuf[pl.ds(j * LANES, LANES)] = z
        cumb[pl.ds(j * LANES, LANES)] = z
        return 0
    lax.fori_loop(0, NVREG, zfill, 0)
    pltpu.sync_copy(susc_hbm.at[pl.ds(s0, SL)], suscb)
    pltpu.sync_copy(zbuf, accg.at[pl.ds(s0, SL)])
    pltpu.sync_copy(betas_hbm, bbuf)

    def ofill(j, _):
        e2a[pl.ds(j * LANES, LANES)] = jnp.ones((LANES,), jnp.float32)
        return 0
    lax.fori_loop(0, CE // LANES, ofill, 0)
    plsc.subcore_barrier()

    d1 = pltpu.async_copy(dst_hbm.at[pl.ds(eb0, CE)], e1a, sem1)
    for c in range(NCHUNK):
        b = c % 2
        d1.wait()
        if c + 1 < NCHUNK:
            d1 = pltpu.async_copy(dst_hbm.at[pl.ds(eb0 + (c + 1) * CE, CE)],
                                  (e1a, e1b)[1 - b], sem1)
        pltpu.sync_copy(e2a, accg.at[(e1a, e1b)[b]], add=True)
    plsc.subcore_barrier()
    pltpu.sync_copy(accg.at[pl.ds(s0, SL)], na)
    pltpu.sync_copy(na, partg_hbm.at[pl.ds(my_part + s0, SL)])
    plsc.subcore_barrier()
    cross_core_sync()
    pltpu.sync_copy(partg_hbm.at[pl.ds(other_part + s0, SL)], nb)

    def invloop(j, _):
        sl = pl.ds(j * LANES, LANES)
        invb[sl] = 1.0 / jnp.maximum(na[sl] + nb[sl], 1.0)
        return 0
    lax.fori_loop(0, NVREG, invloop, 0)
    plsc.subcore_barrier()

    def step(t, _):
        pa0 = pltpu.async_copy(idx1_hbm.at[pl.ds(eb0, CE)], e0a, sem0)
        pa1 = pltpu.async_copy(dst_hbm.at[pl.ds(eb0, CE)], e1a, sem1)

        pltpu.sync_copy(trans_hbm.at[pl.ds(t * NP + s0, SL)], na)
        b0 = bbuf[0]
        b1 = bbuf[1]
        b2 = bbuf[2]
        def e1loop(j, _):
            sl = pl.ds(j * LANES, LANES)
            v = na[sl] + 0.3 * cumb[sl]
            na[sl] = b0 * v
            nb[sl] = b1 * v
            e2a[sl] = b2 * v
            return 0
        lax.fori_loop(0, NVREG, e1loop, 0)
        dz = pltpu.async_copy(zbuf, accg.at[pl.ds(s0, SL)], ssem)
        d0 = pltpu.async_copy(na, trans3.at[pl.ds(s0, SL)], esem0)
        d1 = pltpu.async_copy(nb, trans3.at[pl.ds(NP + s0, SL)], esem1)
        d2 = pltpu.async_copy(e2a.at[pl.ds(0, SL)],
                              trans3.at[pl.ds(2 * NP + s0, SL)], gsem)
        dz.wait()
        d0.wait()
        d1.wait()
        d2.wait()
        plsc.subcore_barrier()

        edge_pass(idx1_hbm, dst_hbm, trans3, accg, pa0, pa1)
        plsc.subcore_barrier()

        pb0 = pltpu.async_copy(dst_hbm.at[pl.ds(eb0, CE)], e0a, sem0)
        pb1 = pltpu.async_copy(src_hbm.at[pl.ds(eb0, CE)], e1a, sem1)

        pltpu.sync_copy(accg.at[pl.ds(s0, SL)], na)
        pltpu.sync_copy(na, partg_hbm.at[pl.ds(my_part + s0, SL)])
        plsc.subcore_barrier()
        cross_core_sync()
        pltpu.sync_copy(partg_hbm.at[pl.ds(other_part + s0, SL)], nb)

        def e2loop(j, _):
            sl = pl.ds(j * LANES, LANES)
            na[sl] = (na[sl] + nb[sl]) * invb[sl]
            return 0
        lax.fori_loop(0, NVREG, e2loop, 0)
        dg = pltpu.async_copy(na, accg.at[pl.ds(s0, SL)], gsem)
        dp = pltpu.async_copy(zbuf, accp.at[pl.ds(s0, SL)], ssem)
        dg.wait()
        dp.wait()
        plsc.subcore_barrier()

        edge_pass(dst_hbm, src_hbm, accg, accp, pb0, pb1)
        plsc.subcore_barrier()

        pltpu.sync_copy(accp.at[pl.ds(s0, SL)], na)
        pltpu.sync_copy(na, partp_hbm.at[pl.ds(my_part + s0, SL)])
        plsc.subcore_barrier()
        cross_core_sync()
        pltpu.sync_copy(partp_hbm.at[pl.ds(other_part + s0, SL)], nb)

        def e3loop(j, _):
            sl = pl.ds(j * LANES, LANES)
            p = (1.0 - jnp.exp(-(na[sl] + nb[sl]))) * suscb[sl]
            nb[sl] = p
            suscb[sl] = suscb[sl] - p
            cumb[sl] = cumb[sl] + p
            return 0
        lax.fori_loop(0, NVREG, e3loop, 0)

        @pl.when(cid == 0)
        def _():
            pltpu.sync_copy(nb, out_hbm.at[pl.ds(t * NP + s0, SL)])
        return 0
    lax.fori_loop(0, T, step, 0)


def kernel(susceptibilities, transmissions, betas, edge_index, edge_type):
    src = edge_index[0]
    dst = edge_index[1]

    susc_p = jnp.pad(susceptibilities, (0, NP - N))
    trans_p = jnp.pad(transmissions, ((0, 0), (0, NP - N)))

    npad = EP - E
    pad_slot = N + (jnp.arange(npad, dtype=jnp.int32) % NPAD_SLOTS)
    src_p = jnp.concatenate([src, pad_slot])
    dst_p = jnp.concatenate([dst, pad_slot])
    type_p = jnp.concatenate([edge_type, jnp.zeros((npad,), jnp.int32)])

    idx1 = src_p + NP * type_p

    betas_rep = jnp.broadcast_to(betas[:, None], (NTYPES, LANES))

    mesh = plsc.VectorSubcoreMesh(
        core_axis_name="c", subcore_axis_name="s", num_cores=NCORES)
    out, _, _ = pl.kernel(
        _sc_kernel_body,
        out_type=(
            jax.ShapeDtypeStruct((T * NP,), jnp.float32),
            jax.ShapeDtypeStruct((NCORES * NP,), jnp.float32),
            jax.ShapeDtypeStruct((NCORES * NP,), jnp.float32),
        ),
        mesh=mesh,
        scratch_types=[
            pltpu.VMEM_SHARED((NTYPES * NP,), jnp.float32),
            pltpu.VMEM_SHARED((NP,), jnp.float32),
            pltpu.VMEM_SHARED((NP,), jnp.float32),
            pltpu.VMEM((SL,), jnp.float32),
            pltpu.VMEM((SL,), jnp.float32),
            pltpu.VMEM((SL,), jnp.float32),
            pltpu.VMEM((SL,), jnp.float32),
            pltpu.VMEM((SL,), jnp.float32),
            pltpu.VMEM((SL,), jnp.float32),
            pltpu.VMEM((NTYPES, LANES), jnp.float32),
            pltpu.VMEM((CE,), jnp.int32),
            pltpu.VMEM((CE,), jnp.int32),
            pltpu.VMEM((CE,), jnp.int32),
            pltpu.VMEM((CE,), jnp.int32),
            pltpu.VMEM((CE,), jnp.float32),
            pltpu.VMEM((CE,), jnp.float32),
            pltpu.SemaphoreType.DMA,
            pltpu.SemaphoreType.DMA,
            pltpu.SemaphoreType.DMA,
            pltpu.SemaphoreType.DMA,
            pltpu.SemaphoreType.DMA,
            pltpu.SemaphoreType.DMA,
            pltpu.SemaphoreType.REGULAR,
        ],
    )(susc_p, trans_p.reshape(-1), betas_rep, idx1, src_p, dst_p)
    return out.reshape(T, NP)[:, :N]


# speedup vs baseline: 1.0837x; 1.0837x over previous
import jax
import jax.numpy as jnp
from jax import lax
from jax.experimental import pallas as pl
from jax.experimental.pallas import tpu as pltpu
from jax.experimental.pallas import tpu_sc as plsc

N = 100000
E = 3200000
T = 10
NTYPES = 3

LANES = 16
NTILES = 16
NCORES = 2
NP = 100096
SL = NP // NTILES
NVREG = SL // LANES
RPT = 784
CR = 56
NCHUNK = RPT // CR
ER = NCORES * NTILES * RPT
EP = ER * 128
CE = CR * 128
ET = RPT * 128
NPAD_SLOTS = NP - N


def _sc_kernel_body(susc_hbm, trans_hbm, betas_hbm, idx1_hbm, src_hbm,
                    dst_hbm,
                    out_hbm, partg_hbm, partp_hbm,
                    trans3, accg, accp,
                    suscb, cumb, invb, zbuf, na, nb, bbuf,
                    e0a, e0b, e1a, e1b, e2a, e2b,
                    sem0, sem1, gsem, ssem, esem0, esem1, xsem):
    cid = lax.axis_index("c")
    tid = lax.axis_index("s")
    s0 = tid * SL
    wid = cid * NTILES + tid
    eb0 = wid * ET
    my_part = cid * NP
    other_part = (1 - cid) * NP

    def cross_core_sync():
        pltpu.semaphore_signal(xsem, 1, core_index=1 - cid)
        pltpu.semaphore_wait(xsem, 1)

    def edge_pass(gidx_hbm, sidx_hbm, table, acc, ld0=None, ld1=None):
        ebufs0 = (e0a, e0b)
        ebufs1 = (e1a, e1b)
        ebufs2 = (e2a, e2b)
        if ld0 is None:
            ld0 = pltpu.async_copy(gidx_hbm.at[pl.ds(eb0, CE)], e0a, sem0)
            ld1 = pltpu.async_copy(sidx_hbm.at[pl.ds(eb0, CE)], e1a, sem1)
        ld0.wait()
        g = pltpu.async_copy(table.at[e0a], e2a, gsem)
        ld1.wait()
        ld0n = ld1n = None
        if NCHUNK > 1:
            rr = eb0 + CE
            ld0n = pltpu.async_copy(gidx_hbm.at[pl.ds(rr, CE)], e0b, sem0)
            ld1n = pltpu.async_copy(sidx_hbm.at[pl.ds(rr, CE)], e1b, sem1)
        for c in range(NCHUNK):
            b = c % 2
            g.wait()
            s = pltpu.async_copy(ebufs2[b], acc.at[ebufs1[b]], ssem,
                                 add=True)
            if c + 1 < NCHUNK:
                ld0n.wait()
                g = pltpu.async_copy(table.at[ebufs0[1 - b]],
                                     ebufs2[1 - b], gsem)
                if c + 2 < NCHUNK:
                    rr = eb0 + (c + 2) * CE
                    ld0n = pltpu.async_copy(gidx_hbm.at[pl.ds(rr, CE)],
                                            ebufs0[b], sem0)
            s.wait()
            if c + 1 < NCHUNK:
                ld1n.wait()
                if c + 2 < NCHUNK:
                    rr = eb0 + (c + 2) * CE
                    ld1n = pltpu.async_copy(sidx_hbm.at[pl.ds(rr, CE)],
                                            ebufs1[b], sem1)

    def zfill(j, _):
        z = jnp.zeros((LANES,), jnp.float32)
        zbuf[pl.ds(j * LANES, LANES)] = z
        cumb[pl.ds(j * LANES, LANES)] = z
        return 0
    lax.fori_loop(0, NVREG, zfill, 0)
    pltpu.sync_copy(susc_hbm.at[pl.ds(s0, SL)], suscb)
    pltpu.sync_copy(zbuf, accg.at[pl.ds(s0, SL)])
    pltpu.sync_copy(betas_hbm, bbuf)

    def ofill(j, _):
        e2a[pl.ds(j * LANES, LANES)] = jnp.ones((LANES,), jnp.float32)
        return 0
    lax.fori_loop(0, CE // LANES, ofill, 0)
    plsc.subcore_barrier()

    d1 = pltpu.async_copy(dst_hbm.at[pl.ds(eb0, CE)], e1a, sem1)
    for c in range(NCHUNK):
        b = c % 2
        d1.wait()
        if c + 1 < NCHUNK:
            d1 = pltpu.async_copy(dst_hbm.at[pl.ds(eb0 + (c + 1) * CE, CE)],
                                  (e1a, e1b)[1 - b], sem1)
        pltpu.sync_copy(e2a, accg.at[(e1a, e1b)[b]], add=True)
    plsc.subcore_barrier()
    pltpu.sync_copy(accg.at[pl.ds(s0, SL)], na)
    pltpu.sync_copy(na, partg_hbm.at[pl.ds(my_part + s0, SL)])
    plsc.subcore_barrier()
    cross_core_sync()
    pltpu.sync_copy(partg_hbm.at[pl.ds(other_part + s0, SL)], nb)

    def invloop(j, _):
        sl = pl.ds(j * LANES, LANES)
        invb[sl] = 1.0 / jnp.maximum(na[sl] + nb[sl], 1.0)
        return 0
    lax.fori_loop(0, NVREG, invloop, 0)
    plsc.subcore_barrier()

    def step(t, _):
        pa0 = pltpu.async_copy(idx1_hbm.at[pl.ds(eb0, CE)], e0a, sem0)
        pa1 = pltpu.async_copy(dst_hbm.at[pl.ds(eb0, CE)], e1a, sem1)

        pltpu.sync_copy(trans_hbm.at[pl.ds(t * NP + s0, SL)], na)
        b0 = bbuf[0]
        b1 = bbuf[1]
        b2 = bbuf[2]
        def e1loop(j, _):
            sl = pl.ds(j * LANES, LANES)
            v = na[sl] + 0.3 * cumb[sl]
            na[sl] = b0 * v
            nb[sl] = b1 * v
            e2a[sl] = b2 * v
            return 0
        lax.fori_loop(0, NVREG, e1loop, 0)
        dz = pltpu.async_copy(zbuf, accg.at[pl.ds(s0, SL)], ssem)
        d0 = pltpu.async_copy(na, trans3.at[pl.ds(s0, SL)], esem0)
        d1 = pltpu.async_copy(nb, trans3.at[pl.ds(NP + s0, SL)], esem1)
        d2 = pltpu.async_copy(e2a.at[pl.ds(0, SL)],
                              trans3.at[pl.ds(2 * NP + s0, SL)], gsem)
        dz.wait()
        d0.wait()
        d1.wait()
        d2.wait()
        plsc.subcore_barrier()

        edge_pass(idx1_hbm, dst_hbm, trans3, accg, pa0, pa1)
        plsc.subcore_barrier()

        pb0 = pltpu.async_copy(dst_hbm.at[pl.ds(eb0, CE)], e0a, sem0)
        pb1 = pltpu.async_copy(src_hbm.at[pl.ds(eb0, CE)], e1a, sem1)

        pltpu.sync_copy(accg.at[pl.ds(s0, SL)], na)
        pltpu.sync_copy(na, partg_hbm.at[pl.ds(my_part + s0, SL)])
        plsc.subcore_barrier()
        cross_core_sync()
        pltpu.sync_copy(partg_hbm.at[pl.ds(other_part + s0, SL)], nb)

        def e2loop(j, _):
            sl = pl.ds(j * LANES, LANES)
            na[sl] = (na[sl] + nb[sl]) * invb[sl]
            return 0
        lax.fori_loop(0, NVREG, e2loop, 0)
        dg = pltpu.async_copy(na, accg.at[pl.ds(s0, SL)], gsem)
        dp = pltpu.async_copy(zbuf, accp.at[pl.ds(s0, SL)], ssem)
        dg.wait()
        dp.wait()
        plsc.subcore_barrier()

        edge_pass(dst_hbm, src_hbm, accg, accp, pb0, pb1)
        plsc.subcore_barrier()

        pltpu.sync_copy(accp.at[pl.ds(s0, SL)], na)
        pltpu.sync_copy(na, partp_hbm.at[pl.ds(my_part + s0, SL)])
        plsc.subcore_barrier()
        cross_core_sync()
        pltpu.sync_copy(partp_hbm.at[pl.ds(other_part + s0, SL)], nb)

        def e3loop(j, _):
            sl = pl.ds(j * LANES, LANES)
            p = (1.0 - jnp.exp(-(na[sl] + nb[sl]))) * suscb[sl]
            nb[sl] = p
            suscb[sl] = suscb[sl] - p
            cumb[sl] = cumb[sl] + p
            return 0
        lax.fori_loop(0, NVREG, e3loop, 0)

        @pl.when(cid == 0)
        def _():
            pltpu.sync_copy(nb, out_hbm.at[pl.ds(t * NP + s0, SL)])
        return 0
    lax.fori_loop(0, T, step, 0)


def kernel(susceptibilities, transmissions, betas, edge_index, edge_type):
    src = edge_index[0]
    dst = edge_index[1]

    susc_p = jnp.pad(susceptibilities, (0, NP - N))
    trans_p = jnp.pad(transmissions, ((0, 0), (0, NP - N)))

    npad = EP - E
    pad_slot = N + (jnp.arange(npad, dtype=jnp.int32) % NPAD_SLOTS)
    src_p = jnp.concatenate([src, pad_slot])
    dst_p = jnp.concatenate([dst, pad_slot])
    type_p = jnp.concatenate([edge_type, jnp.zeros((npad,), jnp.int32)])

    idx1 = src_p + NP * type_p

    betas_rep = jnp.broadcast_to(betas[:, None], (NTYPES, LANES))

    mesh = plsc.VectorSubcoreMesh(
        core_axis_name="c", subcore_axis_name="s", num_cores=NCORES)
    out, _, _ = pl.kernel(
        _sc_kernel_body,
        out_type=(
            jax.ShapeDtypeStruct((T * NP,), jnp.float32),
            jax.ShapeDtypeStruct((NCORES * NP,), jnp.float32),
            jax.ShapeDtypeStruct((NCORES * NP,), jnp.float32),
        ),
        mesh=mesh,
        scratch_types=[
            pltpu.VMEM_SHARED((NTYPES * NP,), jnp.float32),
            pltpu.VMEM_SHARED((NP,), jnp.float32),
            pltpu.VMEM_SHARED((NP,), jnp.float32),
            pltpu.VMEM((SL,), jnp.float32),
            pltpu.VMEM((SL,), jnp.float32),
            pltpu.VMEM((SL,), jnp.float32),
            pltpu.VMEM((SL,), jnp.float32),
            pltpu.VMEM((SL,), jnp.float32),
            pltpu.VMEM((SL,), jnp.float32),
            pltpu.VMEM((NTYPES, LANES), jnp.float32),
            pltpu.VMEM((CE,), jnp.int32),
            pltpu.VMEM((CE,), jnp.int32),
            pltpu.VMEM((CE,), jnp.int32),
            pltpu.VMEM((CE,), jnp.int32),
            pltpu.VMEM((CE,), jnp.float32),
            pltpu.VMEM((CE,), jnp.float32),
            pltpu.SemaphoreType.DMA,
            pltpu.SemaphoreType.DMA,
            pltpu.SemaphoreType.DMA,
            pltpu.SemaphoreType.DMA,
            pltpu.SemaphoreType.DMA,
            pltpu.SemaphoreType.DMA,
            pltpu.SemaphoreType.REGULAR,
        ],
    )(susc_p, trans_p.reshape(-1), betas_rep, idx1, src_p, dst_p)
    return out.reshape(T, NP)[:, :N]
